# baseline (device time: 916376 ns/iter reference)
import jax
import jax.numpy as jnp
from jax import lax
from jax.experimental import pallas as pl
from jax.experimental.pallas import tpu as pltpu

N_DEV = 32
SEQ = 256
D = 1024
HEADS = 8
DH = 128
SCALE = 0.08838834764831843


def kernel(x, Wq, Wo, Wk, Wv):
    def body(x_ref, wq_ref, wo_ref, wk_ref, wv_ref, out_ref,
             msg_ref, send_sems, recv_sems, credit_sem):
        my = lax.axis_index("i")
        left = lax.rem(my - 1 + N_DEV, N_DEV)
        right = lax.rem(my + 1, N_DEV)

        barrier = pltpu.get_barrier_semaphore()
        pl.semaphore_signal(barrier, inc=1, device_id=(left,),
                            device_id_type=pl.DeviceIdType.MESH)
        pl.semaphore_signal(barrier, inc=1, device_id=(right,),
                            device_id_type=pl.DeviceIdType.MESH)
        pl.semaphore_wait(barrier, 2)

        msg_ref[0, 0] = x_ref[0]

        def compute_partial(xv):
            q = jnp.dot(xv, wq_ref[...])
            k = jnp.dot(xv, wk_ref[...])
            v = jnp.dot(xv, wv_ref[...])
            outs = []
            for h in range(HEADS):
                sl = slice(h * DH, (h + 1) * DH)
                s = jnp.dot(q[:, sl], k[:, sl].T) * SCALE
                m = jnp.max(s, axis=-1, keepdims=True)
                p = jnp.exp(s - m)
                p = p / jnp.sum(p, axis=-1, keepdims=True)
                outs.append(jnp.dot(p, v[:, sl]))
            o = jnp.concatenate(outs, axis=1)
            return jnp.dot(o, wo_ref[...])

        def step(t, carry):
            s = lax.rem(t, 2)
            ns = lax.rem(t + 1, 2)

            recv = pltpu.make_async_remote_copy(
                src_ref=msg_ref.at[s], dst_ref=msg_ref.at[s],
                send_sem=send_sems.at[s], recv_sem=recv_sems.at[s],
                device_id=(left,), device_id_type=pl.DeviceIdType.MESH)

            @pl.when(t > 0)
            def _():
                recv.wait_recv()

            partial = compute_partial(msg_ref[s, 0])

            @pl.when(t == 0)
            def _():
                msg_ref[s, 1] = partial

            @pl.when(t > 0)
            def _():
                msg_ref[s, 1] = msg_ref[s, 1] + partial

            @pl.when(t > 0)
            def _():
                pl.semaphore_wait(credit_sem, 1)

            send = pltpu.make_async_remote_copy(
                src_ref=msg_ref.at[s], dst_ref=msg_ref.at[ns],
                send_sem=send_sems.at[s], recv_sem=recv_sems.at[ns],
                device_id=(right,), device_id_type=pl.DeviceIdType.MESH)
            send.start()
            send.wait_send()

            @pl.when(t < N_DEV - 1)
            def _():
                pl.semaphore_signal(credit_sem, inc=1, device_id=(left,),
                                    device_id_type=pl.DeviceIdType.MESH)
            return carry

        lax.fori_loop(0, N_DEV, step, 0)

        final = pltpu.make_async_remote_copy(
            src_ref=msg_ref.at[0], dst_ref=msg_ref.at[0],
            send_sem=send_sems.at[0], recv_sem=recv_sems.at[0],
            device_id=(left,), device_id_type=pl.DeviceIdType.MESH)
        final.wait_recv()
        out_ref[0] = msg_ref[0, 1]

    out_shape = jax.ShapeDtypeStruct((1, SEQ, D), jnp.float32)
    return pl.pallas_call(
        body,
        out_shape=out_shape,
        in_specs=[pl.BlockSpec(memory_space=pltpu.VMEM)] * 5,
        out_specs=pl.BlockSpec(memory_space=pltpu.VMEM),
        scratch_shapes=[
            pltpu.VMEM((2, 2, SEQ, D), jnp.float32),
            pltpu.SemaphoreType.DMA((2,)),
            pltpu.SemaphoreType.DMA((2,)),
            pltpu.SemaphoreType.REGULAR,
        ],
        compiler_params=pltpu.CompilerParams(collective_id=0),
    )(x, Wq, Wo, Wk, Wv)
